# Initial kernel scaffold; baseline (speedup 1.0000x reference)
import jax
import jax.numpy as jnp
from jax.experimental import pallas as pl


def kernel(feats, node2edge_idx, edge_emb, edge_node_adj, sel, W_prep, W_edge_prep, W_e0, W_n0, W_e1, W_n1):
    raise NotImplementedError("write your pallas kernel here")


# speedup vs baseline: 7.0562x; 7.0562x over previous
import functools

import jax
import jax.numpy as jnp
from jax import lax
from jax.experimental import pallas as pl
from jax.experimental.pallas import tpu as pltpu
from jax.experimental.pallas import tpu_sc as plsc

N = 50000
S = 16
E = N * S // 2
D = 128
ED = 16
K = 8
NK = N * K

BN1 = 2000
NB1 = N // BN1


def _idx_f0_body(n2e_ref, sel_ref, feats_ref, wp_ref, idx_ref, f0_ref):
    sel = sel_ref[...]
    n2e = n2e_ref[...]
    acc = jnp.zeros(sel.shape, jnp.int32)
    for s in range(S):
        acc = jnp.where(sel == s, n2e[:, s:s + 1], acc)
    idx_ref[...] = acc
    f0_ref[...] = jnp.dot(feats_ref[...], wp_ref[...],
                          preferred_element_type=jnp.float32)


def _idx_f0(n2e, sel, feats, wp):
    return pl.pallas_call(
        _idx_f0_body,
        grid=(NB1,),
        in_specs=[
            pl.BlockSpec((BN1, S), lambda i: (i, 0)),
            pl.BlockSpec((BN1, K), lambda i: (i, 0)),
            pl.BlockSpec((BN1, D), lambda i: (i, 0)),
            pl.BlockSpec((D, D), lambda i: (0, 0)),
        ],
        out_specs=[
            pl.BlockSpec((BN1, K), lambda i: (i, 0)),
            pl.BlockSpec((BN1, D), lambda i: (i, 0)),
        ],
        out_shape=[
            jax.ShapeDtypeStruct((N, K), jnp.int32),
            jax.ShapeDtypeStruct((N, D), jnp.float32),
        ],
    )(n2e, sel, feats, wp)


def _wc_body(wp_ref, we0b_ref, wc_ref):
    wc_ref[...] = 0.5 * jnp.dot(wp_ref[...], we0b_ref[...],
                                preferred_element_type=jnp.float32)


def _wc(wp, we0b):
    return pl.pallas_call(
        _wc_body,
        out_shape=jax.ShapeDtypeStruct((D, D), jnp.float32),
    )(wp, we0b)


CH = 640
NCH = NK // CH
QR = CH // 128
_NC = 2
_NS = 16
_NW = _NC * _NS


def _gather_body(idx1_hbm, adja_hbm, adjb_hbm, feats_hbm, emb_hbm,
                 ga_hbm, gb_hbm, gemb_hbm,
                 idx_v, a_v, b_v, rows_v, emb_v, sem):
    wid = lax.axis_index("s") * _NC + lax.axis_index("c")
    nt = (NCH - wid + _NW - 1) // _NW

    def body(t, carry):
        c = wid + t * _NW
        base = c * CH
        pltpu.sync_copy(idx1_hbm.at[pl.ds(base, CH)], idx_v)
        cps = [pltpu.async_copy(adja_hbm.at[idx_v.at[pl.ds(q * 128, 128)]],
                                a_v.at[pl.ds(q * 128, 128)], sem)
               for q in range(QR)]
        cps += [pltpu.async_copy(adjb_hbm.at[idx_v.at[pl.ds(q * 128, 128)]],
                                 b_v.at[pl.ds(q * 128, 128)], sem)
                for q in range(QR)]
        cps += [pltpu.async_copy(emb_hbm.at[idx_v.at[pl.ds(q * 128, 128)]],
                                 emb_v.at[pl.ds(q * 128, 128)], sem)
                for q in range(QR)]
        for cp in cps:
            cp.wait()
        pltpu.sync_copy(emb_v, gemb_hbm.at[pl.ds(base, CH)])
        cps = [pltpu.async_copy(feats_hbm.at[a_v.at[pl.ds(q * 128, 128)]],
                                rows_v.at[pl.ds(q * 128, 128)], sem)
               for q in range(QR)]
        for cp in cps:
            cp.wait()
        pltpu.sync_copy(rows_v, ga_hbm.at[pl.ds(base, CH)])
        cps = [pltpu.async_copy(feats_hbm.at[b_v.at[pl.ds(q * 128, 128)]],
                                rows_v.at[pl.ds(q * 128, 128)], sem)
               for q in range(QR)]
        for cp in cps:
            cp.wait()
        pltpu.sync_copy(rows_v, gb_hbm.at[pl.ds(base, CH)])
        return carry

    lax.fori_loop(0, nt, body, 0)


def _gather_sc(idx1, adja, adjb, feats, emb):
    mesh = plsc.VectorSubcoreMesh(core_axis_name="c", subcore_axis_name="s")
    return pl.kernel(
        _gather_body,
        mesh=mesh,
        compiler_params=pltpu.CompilerParams(use_tc_tiling_on_sc=False),
        out_type=(
            jax.ShapeDtypeStruct((NK, D), jnp.float32),
            jax.ShapeDtypeStruct((NK, D), jnp.float32),
            jax.ShapeDtypeStruct((NK, ED), jnp.float32),
        ),
        scratch_types=[
            pltpu.VMEM((CH,), jnp.int32),
            pltpu.VMEM((CH,), jnp.int32),
            pltpu.VMEM((CH,), jnp.int32),
            pltpu.VMEM((CH, D), jnp.float32),
            pltpu.VMEM((CH, ED), jnp.float32),
            pltpu.SemaphoreType.DMA,
        ],
    )(idx1, adja, adjb, feats, emb)


BN3 = 200
BS3 = BN3 * K
NB3 = N // BN3


def _edge_body(ga_ref, gb_ref, gemb_ref, wep_ref, we0a_ref, wc_ref,
               em0_ref, em1_ref):
    ssum = ga_ref[...] + gb_ref[...]
    e0 = jnp.dot(gemb_ref[...], wep_ref[...],
                 preferred_element_type=jnp.float32)
    eo = jnp.maximum(
        jnp.dot(e0, we0a_ref[...], preferred_element_type=jnp.float32)
        + jnp.dot(ssum, wc_ref[...], preferred_element_type=jnp.float32),
        0.0)
    em0_ref[...] = jnp.sum(e0.reshape(BN3, K, D), axis=1) * (1.0 / K)
    em1_ref[...] = jnp.sum(eo.reshape(BN3, K, D), axis=1) * (1.0 / K)


def _edge(ga, gb, gemb, wep, we0a, wc):
    return pl.pallas_call(
        _edge_body,
        grid=(NB3,),
        in_specs=[
            pl.BlockSpec((BS3, D), lambda i: (i, 0)),
            pl.BlockSpec((BS3, D), lambda i: (i, 0)),
            pl.BlockSpec((BS3, ED), lambda i: (i, 0)),
            pl.BlockSpec((ED, D), lambda i: (0, 0)),
            pl.BlockSpec((D, D), lambda i: (0, 0)),
            pl.BlockSpec((D, D), lambda i: (0, 0)),
        ],
        out_specs=[
            pl.BlockSpec((BN3, D), lambda i: (i, 0)),
            pl.BlockSpec((BN3, D), lambda i: (i, 0)),
        ],
        out_shape=[
            jax.ShapeDtypeStruct((N, D), jnp.float32),
            jax.ShapeDtypeStruct((N, D), jnp.float32),
        ],
    )(ga, gb, gemb, wep, we0a, wc)


BN4 = 2000
NB4 = N // BN4


def _node_body(f0_ref, em0_ref, em1_ref, wn0a_ref, wn0b_ref, wn1a_ref,
               wn1b_ref, out_ref):
    f0 = f0_ref[...]
    f1 = jnp.maximum(
        jnp.dot(f0, wn0a_ref[...], preferred_element_type=jnp.float32)
        + jnp.dot(em0_ref[...], wn0b_ref[...],
                  preferred_element_type=jnp.float32), 0.0)
    f2 = jnp.maximum(
        jnp.dot(f1, wn1a_ref[...], preferred_element_type=jnp.float32)
        + jnp.dot(em1_ref[...], wn1b_ref[...],
                  preferred_element_type=jnp.float32), 0.0)
    out_ref[...] = jnp.concatenate([f1, f2], axis=-1)[None]


def _node(f0, em0, em1, wn0a, wn0b, wn1a, wn1b):
    return pl.pallas_call(
        _node_body,
        grid=(NB4,),
        in_specs=[
            pl.BlockSpec((BN4, D), lambda i: (i, 0)),
            pl.BlockSpec((BN4, D), lambda i: (i, 0)),
            pl.BlockSpec((BN4, D), lambda i: (i, 0)),
            pl.BlockSpec((D, D), lambda i: (0, 0)),
            pl.BlockSpec((D, D), lambda i: (0, 0)),
            pl.BlockSpec((D, D), lambda i: (0, 0)),
            pl.BlockSpec((D, D), lambda i: (0, 0)),
        ],
        out_specs=pl.BlockSpec((1, BN4, 2 * D), lambda i: (0, i, 0)),
        out_shape=jax.ShapeDtypeStruct((1, N, 2 * D), jnp.float32),
    )(f0, em0, em1, wn0a, wn0b, wn1a, wn1b)


def kernel(feats, node2edge_idx, edge_emb, edge_node_adj, sel, W_prep,
           W_edge_prep, W_e0, W_n0, W_e1, W_n1):
    del W_e1
    idx, f0 = _idx_f0(node2edge_idx, sel, feats, W_prep)
    wc = _wc(W_prep, W_e0[D:])
    idx1 = idx.reshape(NK)
    adja = edge_node_adj[:, 0]
    adjb = edge_node_adj[:, 1]
    ga, gb, gemb = _gather_sc(idx1, adja, adjb, feats, edge_emb)
    em0, em1 = _edge(ga, gb, gemb, W_edge_prep, W_e0[:D], wc)
    return _node(f0, em0, em1, W_n0[:D], W_n0[D:], W_n1[:D], W_n1[D:])
